# baseline (speedup 1.0000x reference)
import functools

import jax
import jax.numpy as jnp
from jax import lax
from jax.experimental import pallas as pl
from jax.experimental.pallas import tpu as pltpu
from jax.experimental.pallas import tpu_sc as plsc

_NEG = -3.0e38


def _sig(x):
    return 1.0 / (1.0 + jnp.exp(-x))


def _sc_l0_rows(cls0, bb0, ob0):
    info = plsc.get_sparse_core_info()
    nc, ns = info.num_cores, info.num_subcores

    mesh = plsc.VectorSubcoreMesh(core_axis_name="c", subcore_axis_name="s")

    scratch = [
        pltpu.VMEM((40, 8, 80), jnp.float32),
        pltpu.VMEM((4, 8, 80), jnp.float32),
        pltpu.VMEM((8, 80), jnp.float32),
        pltpu.VMEM((8, 80), jnp.float32),
        pltpu.VMEM((8, 80), jnp.float32),
        pltpu.VMEM((8, 80), jnp.float32),
        pltpu.VMEM((6, 16), jnp.float32),
        pltpu.VMEM_SHARED((5, 3, 8, 80), jnp.float32),
        pltpu.SemaphoreType.DMA,
    ]

    @functools.partial(
        pl.kernel,
        mesh=mesh,
        out_type=jax.ShapeDtypeStruct((20, 6, 16), jnp.float32),
        scratch_types=scratch,
    )
    def body(c0h, b0h, o0h, out_hbm, cls_v, bb_v, ob_v, mp_v, pr_v, c0b_v,
             res_v, shr_v, sem):
        wid = lax.axis_index("s") * nc + lax.axis_index("c")
        lane = lax.iota(jnp.int32, 16)
        q = lax.shift_right_logical(wid, 1) & 1
        lb = lax.shift_right_logical(wid, 2)
        blk = lb * 2 + (wid & 1)
        h0 = blk * 8
        active = wid < 20

        @pl.when(active)
        def _phase1():
            cps = (
                pltpu.async_copy(c0h.at[pl.ds(q * 40, 40), pl.ds(h0, 8), :],
                                 cls_v, sem),
                pltpu.async_copy(b0h.at[:, pl.ds(h0, 8), :], bb_v, sem),
                pltpu.async_copy(o0h.at[pl.ds(h0, 8), :], ob_v, sem),
            )
            for cp in cps:
                cp.wait()

            def prow(hh, _):
                for off in (0, 16, 32, 48, 64):
                    sl = pl.ds(off, 16)
                    m = cls_v[0, hh, sl]
                    for c in range(1, 40):
                        m = jnp.maximum(m, cls_v[c, hh, sl])
                    mp_v[hh, sl] = m
                return _

            lax.fori_loop(0, 8, prow, None)

            @pl.when(q == 0)
            def _():
                pltpu.sync_copy(mp_v, shr_v.at[lb, 0])
                pltpu.sync_copy(cls_v.at[0], shr_v.at[lb, 1])

            @pl.when(q == 1)
            def _():
                pltpu.sync_copy(mp_v, shr_v.at[lb, 2])

        plsc.subcore_barrier()

        @pl.when(active)
        def _phase2():
            @pl.when(q == 0)
            def _():
                pltpu.sync_copy(shr_v.at[lb, 2], pr_v)

            @pl.when(q == 1)
            def _():
                pltpu.sync_copy(shr_v.at[lb, 0], pr_v)

            pltpu.sync_copy(shr_v.at[lb, 1], c0b_v)

            res_v[0, :] = jnp.full((16,), -1.0, jnp.float32)
            res_v[1, :] = jnp.full((16,), 2.0e9, jnp.float32)
            res_v[2, :] = jnp.zeros((16,), jnp.float32)
            res_v[3, :] = jnp.zeros((16,), jnp.float32)
            res_v[4, :] = jnp.zeros((16,), jnp.float32)
            res_v[5, :] = jnp.zeros((16,), jnp.float32)

            def row(hh, _):
                gy = (h0 + hh).astype(jnp.float32)
                rb = (h0 + hh) * 80
                for off in (0, 16, 32, 48, 64):
                    sl = pl.ds(off, 16)
                    c0v = c0b_v[hh, sl]
                    mall = jnp.maximum(mp_v[hh, sl], pr_v[hh, sl])
                    person = c0v >= mall
                    key = jnp.where(person, _sig(ob_v[hh, sl]) * _sig(mall),
                                    jnp.float32(-1.0))
                    gx = (off + lane).astype(jnp.float32)
                    cx = (bb_v[0, hh, sl] + gx) * 8.0
                    cy = (bb_v[1, hh, sl] + gy) * 8.0
                    w_ = jnp.exp(bb_v[2, hh, sl]) * 8.0
                    h_ = jnp.exp(bb_v[3, hh, sl]) * 8.0
                    ilf = (rb + off + lane).astype(jnp.float32)
                    bvv = res_v[0, :]
                    upd = key > bvv
                    res_v[0, :] = jnp.where(upd, key, bvv)
                    res_v[1, :] = jnp.where(upd, ilf, res_v[1, :])
                    res_v[2, :] = jnp.where(upd, cx - w_ * 0.5, res_v[2, :])
                    res_v[3, :] = jnp.where(upd, cy - h_ * 0.5, res_v[3, :])
                    res_v[4, :] = jnp.where(upd, cx + w_ * 0.5, res_v[4, :])
                    res_v[5, :] = jnp.where(upd, cy + h_ * 0.5, res_v[5, :])
                return _

            r0 = q * 4
            lax.fori_loop(r0, r0 + 4, row, None)
            pltpu.sync_copy(res_v, out_hbm.at[blk * 2 + q])

    return body(cls0, bb0, ob0)


def _tc_level(cls, bb, ob, gw, stride, base):
    c0 = cls[0]
    m = jnp.max(cls[1:], axis=0)
    person = c0 >= m
    key = jnp.where(person, _sig(ob) * _sig(jnp.maximum(c0, m)),
                    jnp.float32(-1.0))
    gy = lax.broadcasted_iota(jnp.int32, (gw, gw), 0).astype(jnp.float32)
    gx = lax.broadcasted_iota(jnp.int32, (gw, gw), 1).astype(jnp.float32)
    idx = gy * gw + gx + float(base)
    cx = (bb[0] + gx) * stride
    cy = (bb[1] + gy) * stride
    w_ = jnp.exp(bb[2]) * stride
    h_ = jnp.exp(bb[3]) * stride
    k = jnp.max(key)
    sel = key == k
    i_w = jnp.min(jnp.where(sel, idx, jnp.float32(4.0e9)))
    m2 = sel & (idx == i_w)
    x1 = jnp.max(jnp.where(m2, cx - w_ * 0.5, _NEG))
    y1 = jnp.max(jnp.where(m2, cy - h_ * 0.5, _NEG))
    x2 = jnp.max(jnp.where(m2, cx + w_ * 0.5, _NEG))
    y2 = jnp.max(jnp.where(m2, cy + h_ * 0.5, _NEG))
    return k, i_w, x1, y1, x2, y2


def _tc_l12_body(c1_ref, b1_ref, o1_ref, c2_ref, b2_ref, o2_ref, o_ref):
    k1, i1, a1, b1, c1, d1 = _tc_level(c1_ref[...], b1_ref[...], o1_ref[...],
                                       40, 16.0, 6400)
    k2, i2, a2, b2, c2, d2 = _tc_level(c2_ref[...], b2_ref[...], o2_ref[...],
                                       20, 32.0, 8000)
    take2 = k2 > k1
    vals = (jnp.where(take2, k2, k1), jnp.where(take2, i2, i1),
            jnp.where(take2, a2, a1), jnp.where(take2, b2, b1),
            jnp.where(take2, c2, c1), jnp.where(take2, d2, d1))
    ii = lax.broadcasted_iota(jnp.int32, (1, 8), 1)
    row = jnp.zeros((1, 8), jnp.float32)
    for j, v in enumerate(vals):
        row = jnp.where(ii == j, v, row)
    o_ref[...] = row


def _merge_body(rows_ref, tc_ref, o_ref):
    rows = rows_ref[...]
    tcr = tc_ref[...]
    key = rows[:, 0, :]
    m0 = jnp.max(key)
    idxs = rows[:, 1, :]
    cand = key == m0
    i0 = jnp.min(jnp.where(cand, idxs, jnp.float32(4.0e9)))
    m2 = cand & (idxs == i0)
    x1 = jnp.max(jnp.where(m2, rows[:, 2, :], _NEG))
    y1 = jnp.max(jnp.where(m2, rows[:, 3, :], _NEG))
    x2 = jnp.max(jnp.where(m2, rows[:, 4, :], _NEG))
    y2 = jnp.max(jnp.where(m2, rows[:, 5, :], _NEG))
    kt = tcr[0, 0]
    take_t = kt > m0
    m = jnp.where(take_t, kt, m0)
    x1 = jnp.where(take_t, tcr[0, 2], x1)
    y1 = jnp.where(take_t, tcr[0, 3], y1)
    x2 = jnp.where(take_t, tcr[0, 4], x2)
    y2 = jnp.where(take_t, tcr[0, 5], y2)
    use_real = m >= jnp.float32(0.47)
    ii = lax.broadcasted_iota(jnp.int32, (1, 5), 1)
    synth = jnp.where(ii >= 2, jnp.float32(640.0), jnp.float32(0.0))
    synth = jnp.where(ii == 4, jnp.float32(0.47), synth)
    real = jnp.where(ii == 0, x1, jnp.float32(0.0))
    real = jnp.where(ii == 1, y1, real)
    real = jnp.where(ii == 2, x2, real)
    real = jnp.where(ii == 3, y2, real)
    real = jnp.where(ii == 4, m, real)
    o_ref[...] = jnp.where(use_real, real, synth)


def kernel(cls_score_0, cls_score_1, cls_score_2, bbox_pred_0, bbox_pred_1,
           bbox_pred_2, objectness_0, objectness_1, objectness_2):
    rows = _sc_l0_rows(
        cls_score_0.reshape(80, 80, 80),
        bbox_pred_0.reshape(4, 80, 80),
        objectness_0.reshape(80, 80),
    )
    tcrow = pl.pallas_call(
        _tc_l12_body,
        out_shape=jax.ShapeDtypeStruct((1, 8), jnp.float32),
    )(
        cls_score_1.reshape(80, 40, 40),
        bbox_pred_1.reshape(4, 40, 40),
        objectness_1.reshape(40, 40),
        cls_score_2.reshape(80, 20, 20),
        bbox_pred_2.reshape(4, 20, 20),
        objectness_2.reshape(20, 20),
    )
    return pl.pallas_call(
        _merge_body,
        out_shape=jax.ShapeDtypeStruct((1, 5), jnp.float32),
    )(rows, tcrow)


# speedup vs baseline: 1.0077x; 1.0077x over previous
import functools

import jax
import jax.numpy as jnp
from jax import lax
from jax.experimental import pallas as pl
from jax.experimental.pallas import tpu as pltpu
from jax.experimental.pallas import tpu_sc as plsc

_NEG = -3.0e38


def _sig(x):
    return 1.0 / (1.0 + jnp.exp(-x))


def _sc_l0_rows(cls0, ob0):
    info = plsc.get_sparse_core_info()
    nc, ns = info.num_cores, info.num_subcores

    mesh = plsc.VectorSubcoreMesh(core_axis_name="c", subcore_axis_name="s")

    scratch = [
        pltpu.VMEM((80, 8, 80), jnp.float32),
        pltpu.VMEM((8, 80), jnp.float32),
        pltpu.VMEM((2, 16), jnp.float32),
        pltpu.SemaphoreType.DMA,
    ]

    @functools.partial(
        pl.kernel,
        mesh=mesh,
        out_type=jax.ShapeDtypeStruct((20, 2, 16), jnp.float32),
        scratch_types=scratch,
    )
    def body(c0h, o0h, out_hbm, cls_v, ob_v, res_v, sem):
        wid = lax.axis_index("s") * nc + lax.axis_index("c")
        lane = lax.iota(jnp.int32, 16)

        @pl.when(wid < 20)
        def _l0():
            blk = lax.shift_right_logical(wid, 1)
            h0 = blk * 8
            r0 = (wid & 1) * 4
            cps = (
                pltpu.async_copy(c0h.at[:, pl.ds(h0, 8), :], cls_v, sem),
                pltpu.async_copy(o0h.at[pl.ds(h0, 8), :], ob_v, sem),
            )
            for cp in cps:
                cp.wait()
            res_v[0, :] = jnp.full((16,), -1.0, jnp.float32)
            res_v[1, :] = jnp.full((16,), 2.0e9, jnp.float32)

            def row(hh, _):
                rb = (h0 + hh) * 80
                for off in (0, 16, 32, 48, 64):
                    sl = pl.ds(off, 16)
                    c0v = cls_v[0, hh, sl]
                    m = cls_v[1, hh, sl]
                    for c in range(2, 80):
                        m = jnp.maximum(m, cls_v[c, hh, sl])
                    person = c0v >= m
                    mall = jnp.maximum(c0v, m)
                    key = jnp.where(person, _sig(ob_v[hh, sl]) * _sig(mall),
                                    jnp.float32(-1.0))
                    ilf = (rb + off + lane).astype(jnp.float32)
                    bvv = res_v[0, :]
                    upd = key > bvv
                    res_v[0, :] = jnp.where(upd, key, bvv)
                    res_v[1, :] = jnp.where(upd, ilf, res_v[1, :])
                return _

            lax.fori_loop(r0, r0 + 4, row, None)
            pltpu.sync_copy(res_v, out_hbm.at[wid])

    return body(cls0, ob0)


def _tc_level(cls, bb, ob, gw, stride, base):
    c0 = cls[0]
    m = jnp.max(cls[1:], axis=0)
    person = c0 >= m
    key = jnp.where(person, _sig(ob) * _sig(jnp.maximum(c0, m)),
                    jnp.float32(-1.0))
    gy = lax.broadcasted_iota(jnp.int32, (gw, gw), 0).astype(jnp.float32)
    gx = lax.broadcasted_iota(jnp.int32, (gw, gw), 1).astype(jnp.float32)
    idx = gy * gw + gx + float(base)
    cx = (bb[0] + gx) * stride
    cy = (bb[1] + gy) * stride
    w_ = jnp.exp(bb[2]) * stride
    h_ = jnp.exp(bb[3]) * stride
    k = jnp.max(key)
    sel = key == k
    i_w = jnp.min(jnp.where(sel, idx, jnp.float32(4.0e9)))
    m2 = sel & (idx == i_w)
    x1 = jnp.max(jnp.where(m2, cx - w_ * 0.5, _NEG))
    y1 = jnp.max(jnp.where(m2, cy - h_ * 0.5, _NEG))
    x2 = jnp.max(jnp.where(m2, cx + w_ * 0.5, _NEG))
    y2 = jnp.max(jnp.where(m2, cy + h_ * 0.5, _NEG))
    return k, i_w, x1, y1, x2, y2


def _tc_l12_body(c1_ref, b1_ref, o1_ref, c2_ref, b2_ref, o2_ref, o_ref):
    k1, i1, a1, b1, c1, d1 = _tc_level(c1_ref[...], b1_ref[...], o1_ref[...],
                                       40, 16.0, 6400)
    k2, i2, a2, b2, c2, d2 = _tc_level(c2_ref[...], b2_ref[...], o2_ref[...],
                                       20, 32.0, 8000)
    take2 = k2 > k1
    vals = (jnp.where(take2, k2, k1), jnp.where(take2, i2, i1),
            jnp.where(take2, a2, a1), jnp.where(take2, b2, b1),
            jnp.where(take2, c2, c1), jnp.where(take2, d2, d1))
    ii = lax.broadcasted_iota(jnp.int32, (1, 8), 1)
    row = jnp.zeros((1, 8), jnp.float32)
    for j, v in enumerate(vals):
        row = jnp.where(ii == j, v, row)
    o_ref[...] = row


def _merge_body(rows_ref, tc_ref, bb_ref, o_ref):
    rows = rows_ref[...]
    tcr = tc_ref[...]
    bb = bb_ref[...]
    key = rows[:, 0, :]
    m0 = jnp.max(key)
    idxs = rows[:, 1, :]
    cand = key == m0
    i0 = jnp.min(jnp.where(cand, idxs, jnp.float32(4.0e9)))
    i0s = jnp.where(m0 > 0.0, i0, jnp.float32(0.0))
    gy2 = lax.broadcasted_iota(jnp.int32, (80, 80), 0)
    gx2 = lax.broadcasted_iota(jnp.int32, (80, 80), 1)
    idx2 = (gy2 * 80 + gx2).astype(jnp.float32)
    sel = idx2 == i0s
    b0 = jnp.max(jnp.where(sel, bb[0], _NEG))
    b1 = jnp.max(jnp.where(sel, bb[1], _NEG))
    b2 = jnp.max(jnp.where(sel, bb[2], _NEG))
    b3 = jnp.max(jnp.where(sel, bb[3], _NEG))
    gxs = jnp.max(jnp.where(sel, gx2.astype(jnp.float32), _NEG))
    gys = jnp.max(jnp.where(sel, gy2.astype(jnp.float32), _NEG))
    cx = (b0 + gxs) * 8.0
    cy = (b1 + gys) * 8.0
    w_ = jnp.exp(b2) * 8.0
    h_ = jnp.exp(b3) * 8.0
    x1 = cx - w_ * 0.5
    y1 = cy - h_ * 0.5
    x2 = cx + w_ * 0.5
    y2 = cy + h_ * 0.5
    kt = tcr[0, 0]
    take_t = kt > m0
    m = jnp.where(take_t, kt, m0)
    x1 = jnp.where(take_t, tcr[0, 2], x1)
    y1 = jnp.where(take_t, tcr[0, 3], y1)
    x2 = jnp.where(take_t, tcr[0, 4], x2)
    y2 = jnp.where(take_t, tcr[0, 5], y2)
    use_real = m >= jnp.float32(0.47)
    ii = lax.broadcasted_iota(jnp.int32, (1, 5), 1)
    synth = jnp.where(ii >= 2, jnp.float32(640.0), jnp.float32(0.0))
    synth = jnp.where(ii == 4, jnp.float32(0.47), synth)
    real = jnp.where(ii == 0, x1, jnp.float32(0.0))
    real = jnp.where(ii == 1, y1, real)
    real = jnp.where(ii == 2, x2, real)
    real = jnp.where(ii == 3, y2, real)
    real = jnp.where(ii == 4, m, real)
    o_ref[...] = jnp.where(use_real, real, synth)


def kernel(cls_score_0, cls_score_1, cls_score_2, bbox_pred_0, bbox_pred_1,
           bbox_pred_2, objectness_0, objectness_1, objectness_2):
    rows = _sc_l0_rows(
        cls_score_0.reshape(80, 80, 80),
        objectness_0.reshape(80, 80),
    )
    tcrow = pl.pallas_call(
        _tc_l12_body,
        out_shape=jax.ShapeDtypeStruct((1, 8), jnp.float32),
    )(
        cls_score_1.reshape(80, 40, 40),
        bbox_pred_1.reshape(4, 40, 40),
        objectness_1.reshape(40, 40),
        cls_score_2.reshape(80, 20, 20),
        bbox_pred_2.reshape(4, 20, 20),
        objectness_2.reshape(20, 20),
    )
    return pl.pallas_call(
        _merge_body,
        out_shape=jax.ShapeDtypeStruct((1, 5), jnp.float32),
    )(rows, tcrow, bbox_pred_0.reshape(4, 80, 80))
